# baseline (speedup 1.0000x reference)
import jax
import jax.numpy as jnp
from jax import lax
from jax.experimental import pallas as pl
from jax.experimental.pallas import tpu as pltpu
from jax.experimental.pallas import tpu_sc as plsc

NUM_CORES = 2
NUM_SUBCORES = 16
NUM_WORKERS = NUM_CORES * NUM_SUBCORES
LANES = 16
BATCH = 16384
DIM = 32
BPW = BATCH // NUM_WORKERS
GROUPS = BPW // LANES


def _bias_body(uid_hbm, mid_hbm, ubias_hbm, mbias_hbm, w_hbm, b_hbm, out_hbm,
               uid_v, mid_v, ub_v, mb_v, w_v, b_v, out_v, sem_ub, sem_mb):
    wid = lax.axis_index("s") * NUM_CORES + lax.axis_index("c")
    base = wid * BPW

    pltpu.sync_copy(uid_hbm.at[pl.ds(base, BPW)], uid_v)
    pltpu.sync_copy(mid_hbm.at[pl.ds(base, BPW)], mid_v)

    cub = pltpu.async_copy(ubias_hbm.at[uid_v], ub_v, sem_ub)
    cmb = pltpu.async_copy(mbias_hbm.at[mid_v], mb_v, sem_mb)

    pltpu.sync_copy(w_hbm, w_v)
    pltpu.sync_copy(b_hbm, b_v)

    s = w_v[pl.ds(0, LANES)] + w_v[pl.ds(LANES, LANES)]
    w_tot = s[0]
    for i in range(1, LANES):
        w_tot = w_tot + s[i]
    out_bias = b_v[pl.ds(0, LANES)][0]

    cub.wait()
    cmb.wait()

    def group(g, carry):
        gbase = g * LANES
        bp = (ub_v[pl.ds(gbase, LANES)] + mb_v[pl.ds(gbase, LANES)]) * w_tot + out_bias
        out_v[pl.ds(gbase, LANES)] = bp
        return carry

    lax.fori_loop(0, GROUPS, group, 0)
    pltpu.sync_copy(out_v, out_hbm.at[pl.ds(base, BPW)])


U_MAIN = 999936
M_MAIN = 99968
U_CHUNKS = [(k * 76928, 76928) for k in range(12)] + [(923136, 76800)]
M_CHUNKS = [(0, 50048), (50048, 49920)]
CH_MAX = 76928


def _make_repack_body(chunks, out_stride):
    steps = [(s, off, ch) for s in range(4) for (off, ch) in chunks]

    nbuf = 4

    def body(tab_ref, out_ref, b0, b1, b2, b3, i0, i1, i2, i3,
             o0, o1, o2, o3, o4, o5, o6, o7):
        bufs, isems = [b0, b1, b2, b3], [i0, i1, i2, i3]
        osems = [o0, o1, o2, o3, o4, o5, o6, o7]
        pend_out = [[] for _ in range(nbuf)]
        ins = [None] * nbuf

        def start_in(i):
            s, off, ch = steps[i]
            p = i % nbuf
            for dsc in pend_out[p]:
                dsc.wait()
            pend_out[p] = []
            ins[p] = pltpu.async_copy(
                tab_ref.at[pl.ds(8 * s, 8), pl.ds(off, ch)],
                bufs[p].at[:, pl.ds(0, ch)], isems[p])

        for j in range(min(nbuf - 1, len(steps))):
            start_in(j)
        for i, (s, off, ch) in enumerate(steps):
            p = i % nbuf
            if i + nbuf - 1 < len(steps):
                start_in(i + nbuf - 1)
            ins[p].wait()
            for r in range(8):
                pend_out[p].append(pltpu.async_copy(
                    bufs[p].at[r, pl.ds(0, ch)],
                    out_ref.at[pl.ds((8 * s + r) * out_stride + off, ch)],
                    osems[p * 2 + (r & 1)]))
        for p in range(nbuf):
            for dsc in pend_out[p]:
                dsc.wait()

    return body


def _repack(table_t, chunks, out_stride):
    return pl.pallas_call(
        _make_repack_body(chunks, out_stride),
        in_specs=[pl.BlockSpec(memory_space=pl.ANY)],
        out_specs=pl.BlockSpec(memory_space=pl.ANY),
        out_shape=jax.ShapeDtypeStruct((DIM * out_stride,), jnp.float32),
        scratch_shapes=(
            [pltpu.VMEM((8, CH_MAX), jnp.float32)] * 4
            + [pltpu.SemaphoreType.DMA] * 12
        ),
    )(table_t)


def _emb_body(uid_hbm, mid_hbm, uflat_hbm, mflat_hbm, urem_hbm, mrem_hbm,
              bp_hbm, w_hbm, out_hbm,
              uid_v, mid_v, uidc_v, midc_v, ut_v, mt_v, urem_v, mrem_v,
              bp_v, w_v, out_v, sem_u, sem_m):
    wid = lax.axis_index("s") * NUM_CORES + lax.axis_index("c")
    base = wid * BPW

    pltpu.sync_copy(uid_hbm.at[pl.ds(base, BPW)], uid_v)
    pltpu.sync_copy(mid_hbm.at[pl.ds(base, BPW)], mid_v)

    def clamp(g, carry):
        gbase = g * LANES
        uidc_v[pl.ds(gbase, LANES)] = jnp.minimum(
            uid_v[pl.ds(gbase, LANES)], U_MAIN - 1)
        midc_v[pl.ds(gbase, LANES)] = jnp.minimum(
            mid_v[pl.ds(gbase, LANES)], M_MAIN - 1)
        return carry

    lax.fori_loop(0, GROUPS, clamp, 0)

    copies = []
    for d in range(DIM):
        copies.append(pltpu.async_copy(
            uflat_hbm.at[pl.ds(d * U_MAIN, U_MAIN)].at[uidc_v],
            ut_v.at[pl.ds(d * BPW, BPW)], sem_u))
        copies.append(pltpu.async_copy(
            mflat_hbm.at[pl.ds(d * M_MAIN, M_MAIN)].at[midc_v],
            mt_v.at[pl.ds(d * BPW, BPW)], sem_m))

    pltpu.sync_copy(bp_hbm.at[pl.ds(base, BPW)], bp_v)
    pltpu.sync_copy(w_hbm, w_v)
    pltpu.sync_copy(urem_hbm, urem_v)
    pltpu.sync_copy(mrem_hbm, mrem_v)
    w0 = w_v[pl.ds(0, LANES)]
    w1 = w_v[pl.ds(LANES, LANES)]

    for d in range(DIM):
        copies[2 * d].wait()
        copies[2 * d + 1].wait()
        w_d = (w0 if d < LANES else w1)[d % LANES]

        if d == 0:
            def dstep0(g, carry):
                gbase = g * LANES
                u_c = ut_v[pl.ds(gbase, LANES)]
                m_c = mt_v[pl.ds(gbase, LANES)]
                out_v[pl.ds(gbase, LANES)] = (
                    bp_v[pl.ds(gbase, LANES)] + (u_c * m_c) * w_d)
                return carry
            lax.fori_loop(0, GROUPS, dstep0, 0)
        else:
            dbase = d * BPW

            def dstep(g, carry, dbase=dbase, w_d=w_d):
                gbase = g * LANES
                u_c = ut_v[pl.ds(dbase + gbase, LANES)]
                m_c = mt_v[pl.ds(dbase + gbase, LANES)]
                out_v[pl.ds(gbase, LANES)] = (
                    out_v[pl.ds(gbase, LANES)] + (u_c * m_c) * w_d)
                return carry
            lax.fori_loop(0, GROUPS, dstep, 0)

    def patch(g, carry):
        gbase = g * LANES
        uj = uid_v[pl.ds(gbase, LANES)]
        mj = mid_v[pl.ds(gbase, LANES)]
        usel = uj >= U_MAIN
        msel = mj >= M_MAIN
        nfix = (plsc.all_reduce_population_count(usel)
                + plsc.all_reduce_population_count(msel))[0]

        @pl.when(nfix > 0)
        def _():
            urb = jnp.maximum(uj - U_MAIN, 0)
            mrb = jnp.maximum(mj - M_MAIN, 0)
            acc2 = bp_v[pl.ds(gbase, LANES)]
            for d in range(DIM):
                u_c = ut_v[pl.ds(d * BPW + gbase, LANES)]
                m_c = mt_v[pl.ds(d * BPW + gbase, LANES)]
                u_r = plsc.load_gather(urem_v, [urb + d * (1000000 - U_MAIN)])
                m_r = plsc.load_gather(mrem_v, [mrb + d * (100000 - M_MAIN)])
                u_c = jnp.where(usel, u_r, u_c)
                m_c = jnp.where(msel, m_r, m_c)
                w_d = (w0 if d < LANES else w1)[d % LANES]
                acc2 = acc2 + (u_c * m_c) * w_d
            out_v[pl.ds(gbase, LANES)] = acc2

        return carry

    lax.fori_loop(0, GROUPS, patch, 0)
    pltpu.sync_copy(out_v, out_hbm.at[pl.ds(base, BPW)])


@jax.jit
def _run(user_ids, movie_tags, user_emb_t, movie_emb_t, user_bias_flat,
         movie_bias_flat, out_w_flat, out_b_pad):
    mesh = plsc.VectorSubcoreMesh(core_axis_name="c", subcore_axis_name="s",
                                  num_cores=NUM_CORES, num_subcores=NUM_SUBCORES)
    bias_part = pl.kernel(
        _bias_body,
        out_type=jax.ShapeDtypeStruct((BATCH,), jnp.float32),
        mesh=mesh,
        scratch_types=[
            pltpu.VMEM((BPW,), jnp.int32),
            pltpu.VMEM((BPW,), jnp.int32),
            pltpu.VMEM((BPW,), jnp.float32),
            pltpu.VMEM((BPW,), jnp.float32),
            pltpu.VMEM((DIM,), jnp.float32),
            pltpu.VMEM((LANES,), jnp.float32),
            pltpu.VMEM((BPW,), jnp.float32),
            pltpu.SemaphoreType.DMA,
            pltpu.SemaphoreType.DMA,
        ],
        compiler_params=pltpu.CompilerParams(needs_layout_passes=False,
                                             use_tc_tiling_on_sc=False),
    )(user_ids, movie_tags, user_bias_flat, movie_bias_flat, out_w_flat,
      out_b_pad)

    uflat = _repack(user_emb_t, U_CHUNKS, U_MAIN)
    mflat = _repack(movie_emb_t, M_CHUNKS, M_MAIN)
    urem = jnp.reshape(user_emb_t[:, U_MAIN:], (-1,))
    mrem = jnp.reshape(movie_emb_t[:, M_MAIN:], (-1,))

    out = pl.kernel(
        _emb_body,
        out_type=jax.ShapeDtypeStruct((BATCH,), jnp.float32),
        mesh=mesh,
        scratch_types=[
            pltpu.VMEM((BPW,), jnp.int32),
            pltpu.VMEM((BPW,), jnp.int32),
            pltpu.VMEM((BPW,), jnp.int32),
            pltpu.VMEM((BPW,), jnp.int32),
            pltpu.VMEM((BPW * DIM,), jnp.float32),
            pltpu.VMEM((BPW * DIM,), jnp.float32),
            pltpu.VMEM((DIM * 64,), jnp.float32),
            pltpu.VMEM((DIM * 32,), jnp.float32),
            pltpu.VMEM((BPW,), jnp.float32),
            pltpu.VMEM((DIM,), jnp.float32),
            pltpu.VMEM((BPW,), jnp.float32),
            pltpu.SemaphoreType.DMA,
            pltpu.SemaphoreType.DMA,
        ],
        compiler_params=pltpu.CompilerParams(needs_layout_passes=False,
                                             use_tc_tiling_on_sc=False),
    )(user_ids, movie_tags, uflat, mflat, urem, mrem, bias_part, out_w_flat)
    return out


def kernel(user_ids, movie_tags, user_emb, movie_emb, user_bias, movie_bias,
           out_w, out_b):
    out = _run(
        user_ids.astype(jnp.int32),
        movie_tags.astype(jnp.int32),
        user_emb.T,
        movie_emb.T,
        jnp.reshape(user_bias, (-1,)),
        jnp.reshape(movie_bias, (-1,)),
        jnp.reshape(out_w, (-1,)),
        jnp.pad(jnp.reshape(out_b, (-1,)), (0, LANES - 1)),
    )
    return jnp.reshape(out, (BATCH, 1))


# speedup vs baseline: 1.0222x; 1.0222x over previous
import jax
import jax.numpy as jnp
from jax import lax
from jax.experimental import pallas as pl
from jax.experimental.pallas import tpu as pltpu
from jax.experimental.pallas import tpu_sc as plsc

NUM_CORES = 2
NUM_SUBCORES = 16
NUM_WORKERS = NUM_CORES * NUM_SUBCORES
LANES = 16
BATCH = 16384
DIM = 32
BPW = BATCH // NUM_WORKERS
GROUPS = BPW // LANES


def _bias_body(uid_hbm, mid_hbm, ubias_hbm, mbias_hbm, w_hbm, b_hbm, out_hbm,
               uid_v, mid_v, ub_v, mb_v, w_v, b_v, out_v, sem_ub, sem_mb):
    wid = lax.axis_index("s") * NUM_CORES + lax.axis_index("c")
    base = wid * BPW

    pltpu.sync_copy(uid_hbm.at[pl.ds(base, BPW)], uid_v)
    pltpu.sync_copy(mid_hbm.at[pl.ds(base, BPW)], mid_v)

    cub = pltpu.async_copy(ubias_hbm.at[uid_v], ub_v, sem_ub)
    cmb = pltpu.async_copy(mbias_hbm.at[mid_v], mb_v, sem_mb)

    pltpu.sync_copy(w_hbm, w_v)
    pltpu.sync_copy(b_hbm, b_v)

    s = w_v[pl.ds(0, LANES)] + w_v[pl.ds(LANES, LANES)]
    w_tot = s[0]
    for i in range(1, LANES):
        w_tot = w_tot + s[i]
    out_bias = b_v[pl.ds(0, LANES)][0]

    cub.wait()
    cmb.wait()

    def group(g, carry):
        gbase = g * LANES
        bp = (ub_v[pl.ds(gbase, LANES)] + mb_v[pl.ds(gbase, LANES)]) * w_tot + out_bias
        out_v[pl.ds(gbase, LANES)] = bp
        return carry

    lax.fori_loop(0, GROUPS, group, 0)
    pltpu.sync_copy(out_v, out_hbm.at[pl.ds(base, BPW)])


U_MAIN = 999936
M_MAIN = 99968
U_CHUNKS = [(k * 76928, 76928) for k in range(12)] + [(923136, 76800)]
M_CHUNKS = [(0, 50048), (50048, 49920)]
CH_MAX = 76928


def _make_repack_body(chunks, out_stride):
    steps = [(s, off, ch) for s in range(4) for (off, ch) in chunks]

    nbuf = 4

    def body(tab_ref, out_ref, b0, b1, b2, b3, i0, i1, i2, i3, o0, o1, o2, o3):
        bufs, isems, osems = [b0, b1, b2, b3], [i0, i1, i2, i3], [o0, o1, o2, o3]
        pend_out = [[] for _ in range(nbuf)]
        ins = [None] * nbuf

        def start_in(i):
            s, off, ch = steps[i]
            p = i % nbuf
            for dsc in pend_out[p]:
                dsc.wait()
            pend_out[p] = []
            ins[p] = pltpu.async_copy(
                tab_ref.at[pl.ds(8 * s, 8), pl.ds(off, ch)],
                bufs[p].at[:, pl.ds(0, ch)], isems[p])

        for j in range(min(nbuf - 1, len(steps))):
            start_in(j)
        for i, (s, off, ch) in enumerate(steps):
            p = i % nbuf
            if i + nbuf - 1 < len(steps):
                start_in(i + nbuf - 1)
            ins[p].wait()
            for r in range(8):
                pend_out[p].append(pltpu.async_copy(
                    bufs[p].at[r, pl.ds(0, ch)],
                    out_ref.at[pl.ds((8 * s + r) * out_stride + off, ch)],
                    osems[p]))
        for p in range(nbuf):
            for dsc in pend_out[p]:
                dsc.wait()

    return body


def _repack(table_t, chunks, out_stride):
    return pl.pallas_call(
        _make_repack_body(chunks, out_stride),
        in_specs=[pl.BlockSpec(memory_space=pl.ANY)],
        out_specs=pl.BlockSpec(memory_space=pl.ANY),
        out_shape=jax.ShapeDtypeStruct((DIM * out_stride,), jnp.float32),
        scratch_shapes=(
            [pltpu.VMEM((8, CH_MAX), jnp.float32)] * 4
            + [pltpu.SemaphoreType.DMA] * 8
        ),
    )(table_t)


def _emb_body(uid_hbm, mid_hbm, uflat_hbm, mflat_hbm, urem_hbm, mrem_hbm,
              bp_hbm, w_hbm, out_hbm,
              uid_v, mid_v, uidc_v, midc_v, ut_v, mt_v, urem_v, mrem_v,
              bp_v, w_v, out_v, sem_u, sem_m):
    wid = lax.axis_index("s") * NUM_CORES + lax.axis_index("c")
    base = wid * BPW

    pltpu.sync_copy(uid_hbm.at[pl.ds(base, BPW)], uid_v)
    pltpu.sync_copy(mid_hbm.at[pl.ds(base, BPW)], mid_v)

    def clamp(g, carry):
        gbase = g * LANES
        uidc_v[pl.ds(gbase, LANES)] = jnp.minimum(
            uid_v[pl.ds(gbase, LANES)], U_MAIN - 1)
        midc_v[pl.ds(gbase, LANES)] = jnp.minimum(
            mid_v[pl.ds(gbase, LANES)], M_MAIN - 1)
        return carry

    lax.fori_loop(0, GROUPS, clamp, 0)

    copies = []
    for d in range(DIM):
        copies.append(pltpu.async_copy(
            uflat_hbm.at[pl.ds(d * U_MAIN, U_MAIN)].at[uidc_v],
            ut_v.at[pl.ds(d * BPW, BPW)], sem_u))
        copies.append(pltpu.async_copy(
            mflat_hbm.at[pl.ds(d * M_MAIN, M_MAIN)].at[midc_v],
            mt_v.at[pl.ds(d * BPW, BPW)], sem_m))

    pltpu.sync_copy(bp_hbm.at[pl.ds(base, BPW)], bp_v)
    pltpu.sync_copy(w_hbm, w_v)
    pltpu.sync_copy(urem_hbm, urem_v)
    pltpu.sync_copy(mrem_hbm, mrem_v)
    w0 = w_v[pl.ds(0, LANES)]
    w1 = w_v[pl.ds(LANES, LANES)]

    for c in copies:
        c.wait()

    def group(g, carry):
        gbase = g * LANES
        acc = bp_v[pl.ds(gbase, LANES)]
        for d in range(DIM):
            u_c = ut_v[pl.ds(d * BPW + gbase, LANES)]
            m_c = mt_v[pl.ds(d * BPW + gbase, LANES)]
            w_d = (w0 if d < LANES else w1)[d % LANES]
            acc = acc + (u_c * m_c) * w_d
        out_v[pl.ds(gbase, LANES)] = acc

        uj = uid_v[pl.ds(gbase, LANES)]
        mj = mid_v[pl.ds(gbase, LANES)]
        usel = uj >= U_MAIN
        msel = mj >= M_MAIN
        nfix = (plsc.all_reduce_population_count(usel)
                + plsc.all_reduce_population_count(msel))[0]

        @pl.when(nfix > 0)
        def _():
            urb = jnp.maximum(uj - U_MAIN, 0)
            mrb = jnp.maximum(mj - M_MAIN, 0)
            acc2 = bp_v[pl.ds(gbase, LANES)]
            for d in range(DIM):
                u_c = ut_v[pl.ds(d * BPW + gbase, LANES)]
                m_c = mt_v[pl.ds(d * BPW + gbase, LANES)]
                u_r = plsc.load_gather(urem_v, [urb + d * (1000000 - U_MAIN)])
                m_r = plsc.load_gather(mrem_v, [mrb + d * (100000 - M_MAIN)])
                u_c = jnp.where(usel, u_r, u_c)
                m_c = jnp.where(msel, m_r, m_c)
                w_d = (w0 if d < LANES else w1)[d % LANES]
                acc2 = acc2 + (u_c * m_c) * w_d
            out_v[pl.ds(gbase, LANES)] = acc2

        return carry

    lax.fori_loop(0, GROUPS, group, 0)
    pltpu.sync_copy(out_v, out_hbm.at[pl.ds(base, BPW)])


@jax.jit
def _run(user_ids, movie_tags, user_emb_t, movie_emb_t, user_bias_flat,
         movie_bias_flat, out_w_flat, out_b_pad):
    mesh = plsc.VectorSubcoreMesh(core_axis_name="c", subcore_axis_name="s",
                                  num_cores=NUM_CORES, num_subcores=NUM_SUBCORES)
    bias_part = pl.kernel(
        _bias_body,
        out_type=jax.ShapeDtypeStruct((BATCH,), jnp.float32),
        mesh=mesh,
        scratch_types=[
            pltpu.VMEM((BPW,), jnp.int32),
            pltpu.VMEM((BPW,), jnp.int32),
            pltpu.VMEM((BPW,), jnp.float32),
            pltpu.VMEM((BPW,), jnp.float32),
            pltpu.VMEM((DIM,), jnp.float32),
            pltpu.VMEM((LANES,), jnp.float32),
            pltpu.VMEM((BPW,), jnp.float32),
            pltpu.SemaphoreType.DMA,
            pltpu.SemaphoreType.DMA,
        ],
        compiler_params=pltpu.CompilerParams(needs_layout_passes=False,
                                             use_tc_tiling_on_sc=False),
    )(user_ids, movie_tags, user_bias_flat, movie_bias_flat, out_w_flat,
      out_b_pad)

    uflat = _repack(user_emb_t, U_CHUNKS, U_MAIN)
    mflat = _repack(movie_emb_t, M_CHUNKS, M_MAIN)
    urem = jnp.reshape(user_emb_t[:, U_MAIN:], (-1,))
    mrem = jnp.reshape(movie_emb_t[:, M_MAIN:], (-1,))

    out = pl.kernel(
        _emb_body,
        out_type=jax.ShapeDtypeStruct((BATCH,), jnp.float32),
        mesh=mesh,
        scratch_types=[
            pltpu.VMEM((BPW,), jnp.int32),
            pltpu.VMEM((BPW,), jnp.int32),
            pltpu.VMEM((BPW,), jnp.int32),
            pltpu.VMEM((BPW,), jnp.int32),
            pltpu.VMEM((BPW * DIM,), jnp.float32),
            pltpu.VMEM((BPW * DIM,), jnp.float32),
            pltpu.VMEM((DIM * 64,), jnp.float32),
            pltpu.VMEM((DIM * 32,), jnp.float32),
            pltpu.VMEM((BPW,), jnp.float32),
            pltpu.VMEM((DIM,), jnp.float32),
            pltpu.VMEM((BPW,), jnp.float32),
            pltpu.SemaphoreType.DMA,
            pltpu.SemaphoreType.DMA,
        ],
        compiler_params=pltpu.CompilerParams(needs_layout_passes=False,
                                             use_tc_tiling_on_sc=False),
    )(user_ids, movie_tags, uflat, mflat, urem, mrem, bias_part, out_w_flat)
    return out


def kernel(user_ids, movie_tags, user_emb, movie_emb, user_bias, movie_bias,
           out_w, out_b):
    out = _run(
        user_ids.astype(jnp.int32),
        movie_tags.astype(jnp.int32),
        user_emb.T,
        movie_emb.T,
        jnp.reshape(user_bias, (-1,)),
        jnp.reshape(movie_bias, (-1,)),
        jnp.reshape(out_w, (-1,)),
        jnp.pad(jnp.reshape(out_b, (-1,)), (0, LANES - 1)),
    )
    return jnp.reshape(out, (BATCH, 1))
